# baseline (speedup 1.0000x reference)
import functools

import jax
import jax.numpy as jnp
from jax import lax
from jax.experimental import pallas as pl
from jax.experimental.pallas import tpu as pltpu
from jax.experimental.pallas import tpu_sc as plsc

_P_SA0 = 0.1
_P_SA1 = 0.1
_G_SA0 = 0.003
_G_SA1 = 3e-06

_M = 16384
_N = 4096
_NC = 2
_NS = 16
_NW = _NC * _NS
_ROWS_W = _M // _NW
_CR = 8
_CN = 2048
_CSTEPS_N = _N // _CN
_STEPS = (_ROWS_W // _CR) * _CSTEPS_N
_LANES = 16


def _saf_chunk(xb, pb, ob):
    for r in range(_CR):
        @plsc.parallel_loop(0, _CN, step=_LANES, unroll=8)
        def _(c):
            sl = pl.ds(c, _LANES)
            pv = pb[r, sl]
            xv = xb[r, sl]
            ov = jnp.where(pv <= jnp.float32(_P_SA0), jnp.float32(_G_SA0), xv)
            ov = jnp.where(pv > jnp.float32(1.0 - _P_SA1), jnp.float32(_G_SA1), ov)
            ob[r, sl] = ov


_mesh = plsc.VectorSubcoreMesh(core_axis_name="c", subcore_axis_name="s")


@functools.partial(
    pl.kernel,
    mesh=_mesh,
    out_type=jax.ShapeDtypeStruct((_M, _N), jnp.float32),
    scratch_types=[
        pltpu.VMEM((2, _CR, _CN), jnp.float32),
        pltpu.VMEM((2, _CR, _CN), jnp.float32),
        pltpu.VMEM((2, _CR, _CN), jnp.float32),
        pltpu.SemaphoreType.DMA((2,)),
        pltpu.SemaphoreType.DMA((2,)),
        pltpu.SemaphoreType.DMA((2,)),
    ],
)
def _saf_sc(x_hbm, p_hbm, o_hbm, xb, pb, ob, lx_sem, lp_sem, st_sem):
    wid = lax.axis_index("s") * _NC + lax.axis_index("c")
    base = wid * _ROWS_W

    def chunk_slice(s):
        row = base + lax.div(s, _CSTEPS_N) * _CR
        col = lax.rem(s, _CSTEPS_N) * _CN
        return (pl.ds(row, _CR), pl.ds(col, _CN))

    def load(s, b):
        sl = chunk_slice(s)
        pltpu.make_async_copy(x_hbm.at[sl[0], sl[1]], xb.at[b], lx_sem.at[b]).start()
        pltpu.make_async_copy(p_hbm.at[sl[0], sl[1]], pb.at[b], lp_sem.at[b]).start()

    load(0, 0)
    load(1, 1)

    def step(s, _):
        b = lax.rem(s, 2)
        sl = chunk_slice(s)
        pltpu.make_async_copy(x_hbm.at[sl[0], sl[1]], xb.at[b], lx_sem.at[b]).wait()
        pltpu.make_async_copy(p_hbm.at[sl[0], sl[1]], pb.at[b], lp_sem.at[b]).wait()

        @pl.when(s >= 2)
        def _():
            sl_prev = chunk_slice(s - 2)
            pltpu.make_async_copy(ob.at[b], o_hbm.at[sl_prev[0], sl_prev[1]],
                                  st_sem.at[b]).wait()

        _saf_chunk(xb.at[b], pb.at[b], ob.at[b])
        pltpu.make_async_copy(ob.at[b], o_hbm.at[sl[0], sl[1]], st_sem.at[b]).start()

        @pl.when(s + 2 < _STEPS)
        def _():
            load(s + 2, b)

        return 0

    lax.fori_loop(0, _STEPS, step, 0)

    for s in range(_STEPS - 2, _STEPS):
        slf = chunk_slice(s)
        pltpu.make_async_copy(ob.at[s % 2], o_hbm.at[slf[0], slf[1]],
                              st_sem.at[s % 2]).wait()


def kernel(input, p_state):
    return _saf_sc(input, p_state)
